# baseline (speedup 1.0000x reference)
import functools

import jax
import jax.numpy as jnp
from jax import lax
from jax.experimental import pallas as pl
from jax.experimental.pallas import tpu as pltpu
from jax.experimental.pallas import tpu_sc as plsc

N_NODES = 10000
N_EDGES = 320000
F = 144
NC, NS = 2, 16
NW = NC * NS
E_PER_W = N_EDGES // NW
CH = 80
NCH = E_PER_W // CH
RT = 624
RT_LAST = N_NODES - 15 * RT

_mesh = plsc.VectorSubcoreMesh(core_axis_name="c", subcore_axis_name="s")


def _zero_rows(buf, nrows, ncolv):
    def body(i, carry):
        for c in range(ncolv):
            buf[i, pl.ds(c * 16, 16)] = jnp.zeros((16,), jnp.float32)
        return carry
    lax.fori_loop(0, nrows, body, 0)


def _per_tile_rows(s, copyfn):
    @pl.when(s < 15)
    def _():
        copyfn(pl.multiple_of(s * RT, 8), RT)

    @pl.when(s == 15)
    def _():
        copyfn(15 * RT, RT_LAST)


@functools.partial(
    pl.kernel,
    mesh=_mesh,
    out_type=jax.ShapeDtypeStruct((NC, N_NODES, 16), jnp.float32),
    compiler_params=pltpu.CompilerParams(use_tc_tiling_on_sc=False),
    scratch_types=[
        pltpu.VMEM_SHARED((N_NODES, 16), jnp.float32),
        pltpu.VMEM((CH, 16), jnp.float32),
        pltpu.VMEM((RT_LAST, 16), jnp.float32),
        pltpu.VMEM((NCH, CH), jnp.int32),
    ],
)
def _sc_degree(dst3_hbm, out_hbm, acc_sh, ones_v, zero_v, dstall):
    c = lax.axis_index("c")
    s = lax.axis_index("s")
    wid = c * NS + s
    pltpu.sync_copy(dst3_hbm.at[wid], dstall)
    def ones_body(i, carry):
        ones_v[i, pl.ds(0, 16)] = jnp.full((16,), 1.0, jnp.float32)
        return carry
    lax.fori_loop(0, CH, ones_body, 0)
    _zero_rows(zero_v, RT_LAST, 1)
    _per_tile_rows(s, lambda r0, n: pltpu.sync_copy(
        zero_v.at[pl.ds(0, n)], acc_sh.at[pl.ds(r0, n)]))
    plsc.subcore_barrier()
    def body(j, carry):
        pltpu.sync_copy(ones_v, acc_sh.at[dstall.at[j]], add=True)
        return carry
    lax.fori_loop(0, NCH, body, 0)
    plsc.subcore_barrier()
    _per_tile_rows(s, lambda r0, n: pltpu.sync_copy(
        acc_sh.at[pl.ds(r0, n)], out_hbm.at[c, pl.ds(r0, n)]))


def _rsqrt16(x):
    i = lax.bitcast_convert_type(x, jnp.int32)
    y = lax.bitcast_convert_type(
        jnp.int32(0x5F3759DF) - lax.shift_right_logical(i, 1), jnp.float32)
    for _ in range(3):
        y = y * (1.5 - 0.5 * x * y * y)
    return y


def _deg_vectors(degbuf, r0):
    lane0 = jnp.zeros((16,), jnp.int32)
    rows = lax.iota(jnp.int32, 16) + r0
    d0 = plsc.load_gather(degbuf, [lane0, rows, lane0])
    d1 = plsc.load_gather(degbuf, [jnp.ones((16,), jnp.int32), rows, lane0])
    deg = d0 + d1 + 1.0
    return deg, 1.0 / deg, _rsqrt16(deg)


def _make_sc_aggregate(pass2):
    @functools.partial(
        pl.kernel,
        mesh=_mesh,
        out_type=(jax.ShapeDtypeStruct((NC, N_NODES, F), jnp.float32),
                  jax.ShapeDtypeStruct((NC, N_NODES, F), jnp.float32)),
        compiler_params=pltpu.CompilerParams(use_tc_tiling_on_sc=False,
                                             needs_layout_passes=False),
        scratch_types=[
            pltpu.VMEM_SHARED((N_NODES, F), jnp.float32),
            pltpu.VMEM((3, CH, F), jnp.float32),
            pltpu.VMEM((2, CH, 16), jnp.float32),
            pltpu.VMEM((CH + 16,), jnp.float32),
            pltpu.VMEM((CH + 16,), jnp.float32),
            pltpu.VMEM((3, CH), jnp.int32),
            pltpu.VMEM((3, CH), jnp.int32),
            pltpu.SemaphoreType.DMA,
            pltpu.SemaphoreType.DMA,
            pltpu.SemaphoreType.DMA,
        ],
    )
    def kern(*refs):
        if pass2:
            (p_hbm, u0_hbm, degp_hbm, src_hbm, dst_hbm, out_hbm, u_hbm,
             acc_sh, rows3, degbuf, disb, invb, srcv3, dstv3,
             gsem, isem, ssem) = refs
        else:
            (x_hbm, degp_hbm, src_hbm, dst_hbm, out_hbm, u_hbm,
             acc_sh, rows3, degbuf, disb, invb, srcv3, dstv3,
             gsem, isem, ssem) = refs
        c = lax.axis_index("c")
        s = lax.axis_index("s")
        wid = c * NS + s
        base = wid * E_PER_W

        def stripe_chunk(r0, n):
            if pass2:
                pltpu.sync_copy(u0_hbm.at[c, pl.ds(r0, n)],
                                rows3.at[0, pl.ds(0, n)])
                pltpu.sync_copy(p_hbm.at[0, pl.ds(r0, n)],
                                rows3.at[1, pl.ds(0, n)])
                pltpu.sync_copy(p_hbm.at[1, pl.ds(r0, n)],
                                rows3.at[2, pl.ds(0, n)])
            else:
                pltpu.sync_copy(x_hbm.at[pl.ds(r0, n)],
                                rows3.at[0, pl.ds(0, n)])
            pltpu.sync_copy(degp_hbm.at[0, pl.ds(r0, n)],
                            degbuf.at[0, pl.ds(0, n)])
            pltpu.sync_copy(degp_hbm.at[1, pl.ds(r0, n)],
                            degbuf.at[1, pl.ds(0, n)])
            for g in range(n // 16):
                deg, inv, dis = _deg_vectors(degbuf, g * 16)
                disb[pl.ds(g * 16, 16)] = dis
                invb[pl.ds(g * 16, 16)] = inv

            def row_body(r, carry):
                dis_s = disb[pl.ds(r, 16)][0]
                if not pass2:
                    for g in range(F // 16):
                        sl = pl.ds(g * 16, 16)
                        rows3[0, r, sl] = rows3[0, r, sl] * dis_s
                else:
                    inv_s = invb[pl.ds(r, 16)][0]
                    sl8 = pl.ds(128, 16)
                    w8 = (rows3[0, r, sl8] + rows3[1, r, sl8]
                          + rows3[2, r, sl8])
                    for g in range(8):
                        sl = pl.ds(g * 16, 16)
                        w = (rows3[0, r, sl] + rows3[1, r, sl]
                             + rows3[2, r, sl])
                        rows3[0, r, sl] = w * inv_s
                    lane = lax.iota(jnp.int32, 16)
                    v8 = (jnp.where(lane == 1, dis_s, 0.0)
                          + jnp.where(lane == 2, dis_s * w8[0], 0.0))
                    rows3[0, r, sl8] = v8
                return carry
            lax.fori_loop(0, n, row_body, 0)
            pltpu.sync_copy(rows3.at[0, pl.ds(0, n)],
                            u_hbm.at[c, pl.ds(r0, n)])

        def stripe(r0, n):
            for k in range(n // CH):
                stripe_chunk(r0 + k * CH, CH)
            rem = n % CH
            if rem:
                stripe_chunk(r0 + (n // CH) * CH, rem)
        _per_tile_rows(s, stripe)

        _zero_rows(rows3.at[0], CH, F // 16)

        def zcopy(r0, n):
            for k in range(n // CH):
                pltpu.sync_copy(rows3.at[0], acc_sh.at[pl.ds(r0 + k * CH, CH)])
            rem = n % CH
            if rem:
                pltpu.sync_copy(rows3.at[0, pl.ds(0, rem)],
                                acc_sh.at[pl.ds(r0 + (n // CH) * CH, rem)])
        _per_tile_rows(s, zcopy)
        plsc.subcore_barrier()

        usrc = u_hbm.at[c]

        def src_slice(j):
            return src_hbm.at[pl.ds(pl.multiple_of(base + j * CH, 8), CH)]

        def dst_slice(j):
            return dst_hbm.at[pl.ds(pl.multiple_of(base + j * CH, 8), CH)]

        def idx_load(j, slot, sync=False):
            if sync:
                pltpu.sync_copy(src_slice(j), srcv3.at[slot])
                pltpu.sync_copy(dst_slice(j), dstv3.at[slot])
            else:
                pltpu.async_copy(src_slice(j), srcv3.at[slot], isem)
                pltpu.async_copy(dst_slice(j), dstv3.at[slot], isem)

        def idx_wait(j, slot):
            pltpu.make_async_copy(src_slice(j), srcv3.at[slot], isem).wait()
            pltpu.make_async_copy(dst_slice(j), dstv3.at[slot], isem).wait()

        def scat_wait():
            pltpu.make_async_copy(rows3.at[0], acc_sh.at[dstv3.at[0]],
                                  ssem).wait()

        idx_load(0, 0, sync=True)
        pltpu.async_copy(usrc.at[srcv3.at[0]], rows3.at[0], gsem)
        idx_load(1, 1)

        def body(j, carry):
            p = lax.rem(j, 3)
            pn = lax.rem(j + 1, 3)
            pnn = lax.rem(j + 2, 3)

            @pl.when(j + 1 < NCH)
            def _():
                idx_wait(j + 1, pn)

                @pl.when(j >= 2)
                def _():
                    scat_wait()
                pltpu.async_copy(usrc.at[srcv3.at[pn]], rows3.at[pn], gsem)
            pltpu.make_async_copy(usrc.at[srcv3.at[p]], rows3.at[p],
                                  gsem).wait()
            pltpu.async_copy(rows3.at[p], acc_sh.at[dstv3.at[p]], ssem,
                             add=True)

            @pl.when(j + 2 < NCH)
            def _():
                idx_load(j + 2, pnn)
            return carry
        lax.fori_loop(0, NCH, body, 0)
        scat_wait()
        scat_wait()
        scat_wait()
        plsc.subcore_barrier()
        _per_tile_rows(s, lambda r0, n: pltpu.sync_copy(
            acc_sh.at[pl.ds(r0, n)], out_hbm.at[c, pl.ds(r0, n)]))

    return kern


_sc_agg_pass1 = _make_sc_aggregate(pass2=False)
_sc_agg_pass2 = _make_sc_aggregate(pass2=True)


_BR = 1000


def _softplus(x):
    return jnp.maximum(x, 0.0) + jnp.log1p(jnp.exp(-jnp.abs(x)))


def _tc_final_body(p2_ref, u1_ref, w1t_ref, w2t_ref, wht_ref,
                   b1_ref, b2_ref, bh_ref,
                   mt_ref, st_ref, mz_ref, sz_ref):
    hp = jax.lax.Precision.HIGHEST
    t1 = jnp.dot(w1t_ref[...], w2t_ref[...], precision=hp)
    wct = jnp.dot(t1, wht_ref[...], precision=hp)
    c1 = jnp.dot(jnp.dot(b1_ref[...], w2t_ref[...], precision=hp),
                 wht_ref[...], precision=hp)
    c0 = jnp.dot(b2_ref[...], wht_ref[...], precision=hp) + bh_ref[...]
    u1 = u1_ref[...]
    w2 = p2_ref[0] + p2_ref[1] + u1
    dis = u1[:, 129:130]
    a1 = u1[:, 130:131]
    z = dis * w2[:, :128]
    pre = jnp.dot(z, wct, precision=hp) + a1 * c1 + c0
    mt_ref[...] = _softplus(pre[:, 0:1])
    st_ref[...] = _softplus(pre[:, 1:2])
    mz_ref[...] = pre[:, 2:66]
    sz_ref[...] = _softplus(pre[:, 66:130])


def _tc_final(p2, u1, w1t, w2t, wht, b1r, b2r, bhr):
    return pl.pallas_call(
        _tc_final_body,
        grid=(N_NODES // _BR,),
        in_specs=[
            pl.BlockSpec((NC, _BR, F), lambda i: (0, i, 0)),
            pl.BlockSpec((_BR, F), lambda i: (i, 0)),
            pl.BlockSpec((128, 500), lambda i: (0, 0)),
            pl.BlockSpec((500, 250), lambda i: (0, 0)),
            pl.BlockSpec((250, 130), lambda i: (0, 0)),
            pl.BlockSpec((1, 500), lambda i: (0, 0)),
            pl.BlockSpec((1, 250), lambda i: (0, 0)),
            pl.BlockSpec((1, 130), lambda i: (0, 0)),
        ],
        out_specs=[
            pl.BlockSpec((_BR, 1), lambda i: (i, 0)),
            pl.BlockSpec((_BR, 1), lambda i: (i, 0)),
            pl.BlockSpec((_BR, 64), lambda i: (i, 0)),
            pl.BlockSpec((_BR, 64), lambda i: (i, 0)),
        ],
        out_shape=(jax.ShapeDtypeStruct((N_NODES, 1), jnp.float32),
                   jax.ShapeDtypeStruct((N_NODES, 1), jnp.float32),
                   jax.ShapeDtypeStruct((N_NODES, 64), jnp.float32),
                   jax.ShapeDtypeStruct((N_NODES, 64), jnp.float32)),
    )(p2, u1, w1t, w2t, wht, b1r, b2r, bhr)


def kernel(data_in, edge_index, W1, b1, W2, b2,
           Wmt, bmt, Wst, bst, Wmz, bmz, Wsz, bsz):
    src = edge_index[0]
    dst = edge_index[1]
    dst3 = dst.reshape(NW, NCH, CH)
    xpad = jnp.concatenate(
        [data_in,
         jnp.ones((N_NODES, 1), jnp.float32),
         jnp.zeros((N_NODES, F - 129), jnp.float32)], axis=1)

    degp = _sc_degree(dst3)
    p1, _u0 = _sc_agg_pass1(xpad, degp, src, dst)
    p2, u1s = _sc_agg_pass2(p1, _u0, degp, src, dst)

    wht = jnp.concatenate([Wmt, Wst, Wmz, Wsz], axis=0).T
    bhr = jnp.concatenate([bmt, bst, bmz, bsz])[None, :]
    return _tc_final(p2, u1s[0], W1.T, W2.T, wht, b1[None, :], b2[None, :], bhr)


# speedup vs baseline: 1.1068x; 1.1068x over previous
import functools

import jax
import jax.numpy as jnp
from jax import lax
from jax.experimental import pallas as pl
from jax.experimental.pallas import tpu as pltpu
from jax.experimental.pallas import tpu_sc as plsc

N_NODES = 10000
N_EDGES = 320000
F = 144
NC, NS = 2, 16
NW = NC * NS
E_PER_W = N_EDGES // NW
CH = 80
NCH = E_PER_W // CH
RT = 624
RT_LAST = N_NODES - 15 * RT

_mesh = plsc.VectorSubcoreMesh(core_axis_name="c", subcore_axis_name="s")


def _zero_rows(buf, nrows, ncolv):
    def body(i, carry):
        for c in range(ncolv):
            buf[i, pl.ds(c * 16, 16)] = jnp.zeros((16,), jnp.float32)
        return carry
    lax.fori_loop(0, nrows, body, 0)


def _per_tile_rows(s, copyfn):
    @pl.when(s < 15)
    def _():
        copyfn(pl.multiple_of(s * RT, 8), RT)

    @pl.when(s == 15)
    def _():
        copyfn(15 * RT, RT_LAST)


@functools.partial(
    pl.kernel,
    mesh=_mesh,
    out_type=jax.ShapeDtypeStruct((NC, N_NODES, 16), jnp.float32),
    compiler_params=pltpu.CompilerParams(use_tc_tiling_on_sc=False),
    scratch_types=[
        pltpu.VMEM_SHARED((N_NODES, 16), jnp.float32),
        pltpu.VMEM((CH, 16), jnp.float32),
        pltpu.VMEM((RT_LAST, 16), jnp.float32),
        pltpu.VMEM((NCH, CH), jnp.int32),
    ],
)
def _sc_degree(dst3_hbm, out_hbm, acc_sh, ones_v, zero_v, dstall):
    c = lax.axis_index("c")
    s = lax.axis_index("s")
    wid = c * NS + s
    pltpu.sync_copy(dst3_hbm.at[wid], dstall)
    def ones_body(i, carry):
        ones_v[i, pl.ds(0, 16)] = jnp.full((16,), 1.0, jnp.float32)
        return carry
    lax.fori_loop(0, CH, ones_body, 0)
    _zero_rows(zero_v, RT_LAST, 1)
    _per_tile_rows(s, lambda r0, n: pltpu.sync_copy(
        zero_v.at[pl.ds(0, n)], acc_sh.at[pl.ds(r0, n)]))
    plsc.subcore_barrier()
    def body(j, carry):
        pltpu.sync_copy(ones_v, acc_sh.at[dstall.at[j]], add=True)
        return carry
    lax.fori_loop(0, NCH, body, 0)
    plsc.subcore_barrier()
    _per_tile_rows(s, lambda r0, n: pltpu.sync_copy(
        acc_sh.at[pl.ds(r0, n)], out_hbm.at[c, pl.ds(r0, n)]))


@functools.partial(
    pl.kernel,
    mesh=_mesh,
    out_type=jax.ShapeDtypeStruct((NC, N_NODES, F), jnp.float32),
    compiler_params=pltpu.CompilerParams(use_tc_tiling_on_sc=False),
    scratch_types=[
        pltpu.VMEM_SHARED((N_NODES, F), jnp.float32),
        pltpu.VMEM((2, CH, F), jnp.float32),
        pltpu.VMEM((NCH, CH), jnp.int32),
        pltpu.VMEM((2, CH), jnp.int32),
        pltpu.SemaphoreType.DMA,
        pltpu.SemaphoreType.DMA,
        pltpu.SemaphoreType.DMA,
    ],
)
def _sc_aggregate(u_hbm, src_hbm, dst3_hbm, out_hbm,
                  acc_sh, rows2, dstall, srcv2, gsem, isem, ssem):
    c = lax.axis_index("c")
    s = lax.axis_index("s")
    wid = c * NS + s
    base = wid * E_PER_W
    pltpu.sync_copy(dst3_hbm.at[wid], dstall)
    _zero_rows(rows2.at[0], CH, F // 16)

    def zcopy(r0, n):
        for k in range(n // CH):
            pltpu.sync_copy(rows2.at[0], acc_sh.at[pl.ds(r0 + k * CH, CH)])
        rem = n % CH
        if rem:
            pltpu.sync_copy(rows2.at[0, pl.ds(0, rem)],
                            acc_sh.at[pl.ds(r0 + (n // CH) * CH, rem)])
    _per_tile_rows(s, zcopy)
    plsc.subcore_barrier()

    def src_slice(j):
        return src_hbm.at[pl.ds(pl.multiple_of(base + j * CH, 8), CH)]

    pltpu.sync_copy(src_slice(0), srcv2.at[0])
    pltpu.async_copy(u_hbm.at[srcv2.at[0]], rows2.at[0], gsem)
    pltpu.async_copy(src_slice(1), srcv2.at[1], isem)

    def scat_wait():
        pltpu.make_async_copy(rows2.at[0], acc_sh.at[dstall.at[0]], ssem).wait()

    def body(j, carry):
        p = lax.rem(j, 2)
        pn = lax.rem(j + 1, 2)

        @pl.when(j + 1 < NCH)
        def _():
            pltpu.make_async_copy(src_slice(j + 1), srcv2.at[pn], isem).wait()

            @pl.when(j >= 1)
            def _():
                scat_wait()
            pltpu.async_copy(u_hbm.at[srcv2.at[pn]], rows2.at[pn], gsem)
        pltpu.make_async_copy(u_hbm.at[srcv2.at[p]], rows2.at[p], gsem).wait()
        pltpu.async_copy(rows2.at[p], acc_sh.at[dstall.at[j]], ssem, add=True)

        @pl.when(j + 2 < NCH)
        def _():
            pltpu.async_copy(src_slice(j + 2), srcv2.at[p], isem)
        return carry
    lax.fori_loop(0, NCH, body, 0)
    scat_wait()
    scat_wait()
    plsc.subcore_barrier()
    _per_tile_rows(s, lambda r0, n: pltpu.sync_copy(
        acc_sh.at[pl.ds(r0, n)], out_hbm.at[c, pl.ds(r0, n)]))


_BR = 1000


def _tc_prep_body(x_ref, degp_ref, out_ref):
    deg = degp_ref[0, :, 0:1] + degp_ref[1, :, 0:1] + 1.0
    dis = lax.rsqrt(deg)
    out_ref[...] = jnp.concatenate(
        [x_ref[...] * dis, dis, jnp.zeros((_BR, F - 129), jnp.float32)], axis=1)


def _tc_prep(x, degp):
    return pl.pallas_call(
        _tc_prep_body,
        grid=(N_NODES // _BR,),
        in_specs=[
            pl.BlockSpec((_BR, 128), lambda i: (i, 0)),
            pl.BlockSpec((NC, _BR, 16), lambda i: (0, i, 0)),
        ],
        out_specs=pl.BlockSpec((_BR, F), lambda i: (i, 0)),
        out_shape=jax.ShapeDtypeStruct((N_NODES, F), jnp.float32),
    )(x, degp)


def _tc_mid_body(p1_ref, u0_ref, degp_ref, out_ref):
    deg = degp_ref[0, :, 0:1] + degp_ref[1, :, 0:1] + 1.0
    dis = lax.rsqrt(deg)
    inv = 1.0 / deg
    w1 = p1_ref[0] + p1_ref[1] + u0_ref[...]
    out_ref[...] = jnp.concatenate(
        [inv * w1[:, :128],
         jnp.zeros((_BR, 1), jnp.float32),
         dis,
         dis * w1[:, 128:129],
         jnp.zeros((_BR, F - 131), jnp.float32)], axis=1)


def _tc_mid(p1, u0, degp):
    return pl.pallas_call(
        _tc_mid_body,
        grid=(N_NODES // _BR,),
        in_specs=[
            pl.BlockSpec((NC, _BR, F), lambda i: (0, i, 0)),
            pl.BlockSpec((_BR, F), lambda i: (i, 0)),
            pl.BlockSpec((NC, _BR, 16), lambda i: (0, i, 0)),
        ],
        out_specs=pl.BlockSpec((_BR, F), lambda i: (i, 0)),
        out_shape=jax.ShapeDtypeStruct((N_NODES, F), jnp.float32),
    )(p1, u0, degp)


def _softplus(x):
    return jnp.maximum(x, 0.0) + jnp.log1p(jnp.exp(-jnp.abs(x)))


def _tc_final_body(p2_ref, u1_ref, w1t_ref, w2t_ref, wht_ref,
                   b1_ref, b2_ref, bh_ref,
                   mt_ref, st_ref, mz_ref, sz_ref):
    hp = jax.lax.Precision.HIGHEST
    t1 = jnp.dot(w1t_ref[...], w2t_ref[...], precision=hp)
    wct = jnp.dot(t1, wht_ref[...], precision=hp)
    c1 = jnp.dot(jnp.dot(b1_ref[...], w2t_ref[...], precision=hp),
                 wht_ref[...], precision=hp)
    c0 = jnp.dot(b2_ref[...], wht_ref[...], precision=hp) + bh_ref[...]
    u1 = u1_ref[...]
    w2 = p2_ref[0] + p2_ref[1] + u1
    dis = u1[:, 129:130]
    a1 = u1[:, 130:131]
    z = dis * w2[:, :128]
    pre = jnp.dot(z, wct, precision=hp) + a1 * c1 + c0
    mt_ref[...] = _softplus(pre[:, 0:1])
    st_ref[...] = _softplus(pre[:, 1:2])
    mz_ref[...] = pre[:, 2:66]
    sz_ref[...] = _softplus(pre[:, 66:130])


def _tc_final(p2, u1, w1t, w2t, wht, b1r, b2r, bhr):
    return pl.pallas_call(
        _tc_final_body,
        grid=(N_NODES // _BR,),
        in_specs=[
            pl.BlockSpec((NC, _BR, F), lambda i: (0, i, 0)),
            pl.BlockSpec((_BR, F), lambda i: (i, 0)),
            pl.BlockSpec((128, 500), lambda i: (0, 0)),
            pl.BlockSpec((500, 250), lambda i: (0, 0)),
            pl.BlockSpec((250, 130), lambda i: (0, 0)),
            pl.BlockSpec((1, 500), lambda i: (0, 0)),
            pl.BlockSpec((1, 250), lambda i: (0, 0)),
            pl.BlockSpec((1, 130), lambda i: (0, 0)),
        ],
        out_specs=[
            pl.BlockSpec((_BR, 1), lambda i: (i, 0)),
            pl.BlockSpec((_BR, 1), lambda i: (i, 0)),
            pl.BlockSpec((_BR, 64), lambda i: (i, 0)),
            pl.BlockSpec((_BR, 64), lambda i: (i, 0)),
        ],
        out_shape=(jax.ShapeDtypeStruct((N_NODES, 1), jnp.float32),
                   jax.ShapeDtypeStruct((N_NODES, 1), jnp.float32),
                   jax.ShapeDtypeStruct((N_NODES, 64), jnp.float32),
                   jax.ShapeDtypeStruct((N_NODES, 64), jnp.float32)),
    )(p2, u1, w1t, w2t, wht, b1r, b2r, bhr)


def kernel(data_in, edge_index, W1, b1, W2, b2,
           Wmt, bmt, Wst, bst, Wmz, bmz, Wsz, bsz):
    src = edge_index[0]
    dst = edge_index[1]
    dst3 = dst.reshape(NW, NCH, CH)

    degp = _sc_degree(dst3)
    u0 = _tc_prep(data_in, degp)
    p1 = _sc_aggregate(u0, src, dst3)
    u1 = _tc_mid(p1, u0, degp)
    p2 = _sc_aggregate(u1, src, dst3)

    wht = jnp.concatenate([Wmt, Wst, Wmz, Wsz], axis=0).T
    bhr = jnp.concatenate([bmt, bst, bmz, bsz])[None, :]
    return _tc_final(p2, u1, W1.T, W2.T, wht, b1[None, :], b2[None, :], bhr)
